# baseline (speedup 1.0000x reference)
import jax
import jax.numpy as jnp
from jax import lax
from jax.experimental import pallas as pl
from jax.experimental.pallas import tpu as pltpu
from jax.experimental.pallas import tpu_sc as plsc

_VOCAB = 50304
_MAX_POS = 8192
_EMBED = 768
_BATCH = 4
_SEQ = 8192

_NC = 2
_NS = 16
_NW = _NC * _NS
_POS_PER_W = _SEQ // _NW
_C = 32
_NCHUNK = _POS_PER_W // _C
_NVREG = _EMBED // 16


def _body(ids_hbm, w_word, w_pos, out_hbm,
          idx_v, rbuf0, rbuf1, posbuf,
          sem_g0, sem_g1, sem_s0, sem_s1, sem_p):
    cid = lax.axis_index("c")
    sid = lax.axis_index("s")
    wid = sid * _NC + cid
    pos0 = wid * _POS_PER_W

    rbufs = (rbuf0, rbuf1)
    gsems = (sem_g0, sem_g1)
    ssems = (sem_s0, sem_s1)

    def gather(b_idx, ds_idx, rb, sem):
        pltpu.async_copy(w_word.at[idx_v.at[b_idx, ds_idx]], rb, sem)

    def wait_bytes(rb, sem):
        pltpu.make_async_copy(w_pos.at[pl.ds(0, _C)], rb, sem).wait()

    pltpu.sync_copy(ids_hbm.at[:, pl.ds(pos0, _POS_PER_W)], idx_v)

    pltpu.async_copy(w_pos.at[pl.ds(pos0, _C)], posbuf.at[0], sem_p.at[0])
    gather(0, pl.ds(0, _C), rbuf0, sem_g0)

    def chunk_body(g, _):
        gm = lax.rem(g, 2)
        gn = lax.rem(g + 1, 2)

        @pl.when(g < _NCHUNK - 1)
        def _prefetch_pos():
            pltpu.async_copy(
                w_pos.at[pl.ds(pos0 + (g + 1) * _C, _C)],
                posbuf.at[gn], sem_p.at[gn])

        pltpu.make_async_copy(
            w_pos.at[pl.ds(0, _C)], posbuf.at[gm], sem_p.at[gm]).wait()

        for b in range(_BATCH):
            cur, nxt = b % 2, (b + 1) % 2

            if b == 0:
                @pl.when(g > 0)
                def _wait_prev_scatter():
                    wait_bytes(rbufs[nxt], ssems[nxt])
            else:
                wait_bytes(rbufs[nxt], ssems[nxt])

            if b < _BATCH - 1:
                gather(b + 1, pl.ds(g * _C, _C), rbufs[nxt], gsems[nxt])
            else:
                @pl.when(g < _NCHUNK - 1)
                def _gather_next_chunk():
                    gather(0, pl.ds((g + 1) * _C, _C), rbufs[nxt], gsems[nxt])

            wait_bytes(rbufs[cur], gsems[cur])

            def add_row(r, carry):
                for j in range(_NVREG):
                    s = pl.ds(j * 16, 16)
                    rbufs[cur][r, s] = rbufs[cur][r, s] + posbuf[gm, r, s]
                return carry

            lax.fori_loop(0, _C, add_row, 0, unroll=False)

            pltpu.async_copy(
                rbufs[cur],
                out_hbm.at[pl.ds(b * _SEQ + pos0 + g * _C, _C)],
                ssems[cur])
        return _

    lax.fori_loop(0, _NCHUNK, chunk_body, 0, unroll=False)

    wait_bytes(rbuf1, sem_s1)


@jax.jit
def _embed(input_ids, w_word, w_pos):
    mesh = plsc.VectorSubcoreMesh(core_axis_name="c", subcore_axis_name="s")
    k = pl.kernel(
        _body,
        out_type=jax.ShapeDtypeStruct((_BATCH * _SEQ, _EMBED), jnp.float32),
        mesh=mesh,
        scratch_types=[
            pltpu.VMEM((_BATCH, _POS_PER_W), jnp.int32),
            pltpu.VMEM((_C, _EMBED), jnp.float32),
            pltpu.VMEM((_C, _EMBED), jnp.float32),
            pltpu.VMEM((2, _C, _EMBED), jnp.float32),
            pltpu.SemaphoreType.DMA,
            pltpu.SemaphoreType.DMA,
            pltpu.SemaphoreType.DMA,
            pltpu.SemaphoreType.DMA,
            pltpu.SemaphoreType.DMA((2,)),
        ],
    )
    return k(input_ids, w_word, w_pos)


def kernel(input_ids, W_word, W_pos):
    ids = input_ids.astype(jnp.int32)
    out = _embed(ids, W_word, W_pos)
    return out.reshape(_BATCH, _SEQ, _EMBED)


# speedup vs baseline: 2.3109x; 2.3109x over previous
import jax
import jax.numpy as jnp
from jax import lax
from jax.experimental import pallas as pl
from jax.experimental.pallas import tpu as pltpu
from jax.experimental.pallas import tpu_sc as plsc

_VOCAB = 50304
_MAX_POS = 8192
_EMBED = 768
_BATCH = 4
_SEQ = 8192

_NC = 2
_NS = 16
_NW = _NC * _NS
_POS_PER_W = _SEQ // _NW
_C = 32
_NCHUNK = _POS_PER_W // _C
_NVREG = _EMBED // 16


def _body(ids_hbm, w_word, w_pos, out_hbm,
          idx_v, rbuf0, rbuf1, posbuf,
          sem_g0, sem_g1, sem_s0, sem_s1):
    cid = lax.axis_index("c")
    sid = lax.axis_index("s")
    wid = sid * _NC + cid
    pos0 = wid * _POS_PER_W

    rbufs = (rbuf0, rbuf1)
    gsems = (sem_g0, sem_g1)
    ssems = (sem_s0, sem_s1)

    def gather(b_idx, off, rb, sem):
        pltpu.async_copy(w_word.at[idx_v.at[b_idx, pl.ds(off, _C)]], rb, sem)

    def wait_tile(rb, sem):
        pltpu.make_async_copy(w_pos.at[pl.ds(0, _C)], rb, sem).wait()

    pltpu.sync_copy(ids_hbm.at[:, pl.ds(pos0, _POS_PER_W)], idx_v)

    pltpu.async_copy(w_pos.at[pl.ds(pos0, _C)], rbuf1, sem_s1)
    gather(0, 0, rbuf0, sem_g0)

    def chunk_body(g, _):
        pltpu.sync_copy(w_pos.at[pl.ds(pos0 + g * _C, _C)], posbuf)

        for b in range(_BATCH):
            cur, nxt = b % 2, 1 - b % 2

            wait_tile(rbufs[nxt], ssems[nxt])

            if b < _BATCH - 1:
                gather(b + 1, g * _C, rbufs[nxt], gsems[nxt])
            else:
                gather(0, lax.rem((g + 1) * _C, _POS_PER_W),
                       rbufs[nxt], gsems[nxt])

            wait_tile(rbufs[cur], gsems[cur])

            def add_row(r, carry):
                for j in range(_NVREG):
                    s = pl.ds(j * 16, 16)
                    rbufs[cur][r, s] = rbufs[cur][r, s] + posbuf[r, s]
                return carry

            lax.fori_loop(0, _C, add_row, 0, unroll=False)

            pltpu.async_copy(
                rbufs[cur],
                out_hbm.at[pl.ds(b * _SEQ + pos0 + g * _C, _C)],
                ssems[cur])
        return _

    lax.fori_loop(0, _NCHUNK, chunk_body, 0, unroll=False)

    wait_tile(rbuf0, sem_g0)
    wait_tile(rbuf1, sem_s1)


@jax.jit
def _embed(input_ids, w_word, w_pos):
    mesh = plsc.VectorSubcoreMesh(core_axis_name="c", subcore_axis_name="s")
    k = pl.kernel(
        _body,
        out_type=jax.ShapeDtypeStruct((_BATCH * _SEQ, _EMBED), jnp.float32),
        mesh=mesh,
        scratch_types=[
            pltpu.VMEM((_BATCH, _POS_PER_W), jnp.int32),
            pltpu.VMEM((_C, _EMBED), jnp.float32),
            pltpu.VMEM((_C, _EMBED), jnp.float32),
            pltpu.VMEM((_C, _EMBED), jnp.float32),
            pltpu.SemaphoreType.DMA,
            pltpu.SemaphoreType.DMA,
            pltpu.SemaphoreType.DMA,
            pltpu.SemaphoreType.DMA,
        ],
    )
    return k(input_ids, w_word, w_pos)


def kernel(input_ids, W_word, W_pos):
    ids = input_ids.astype(jnp.int32)
    out = _embed(ids, W_word, W_pos)
    return out.reshape(_BATCH, _SEQ, _EMBED)


# speedup vs baseline: 2.4047x; 1.0406x over previous
import jax
import jax.numpy as jnp
from jax import lax
from jax.experimental import pallas as pl
from jax.experimental.pallas import tpu as pltpu
from jax.experimental.pallas import tpu_sc as plsc

_VOCAB = 50304
_MAX_POS = 8192
_EMBED = 768
_BATCH = 4
_SEQ = 8192

_NC = 2
_NS = 16
_NW = _NC * _NS
_POS_PER_W = _SEQ // _NW
_C = 32
_NCHUNK = _POS_PER_W // _C
_NVREG = _EMBED // 16


def _body(ids_hbm, w_word, w_pos, out_hbm,
          idx_v, rbuf0, rbuf1, posbuf0, posbuf1,
          sem_g0, sem_g1, sem_s0, sem_s1, sem_p0, sem_p1):
    cid = lax.axis_index("c")
    sid = lax.axis_index("s")
    wid = sid * _NC + cid
    pos0 = wid * _POS_PER_W

    rbufs = (rbuf0, rbuf1)
    gsems = (sem_g0, sem_g1)
    ssems = (sem_s0, sem_s1)

    def gather(b_idx, off, rb, sem):
        pltpu.async_copy(w_word.at[idx_v.at[b_idx, pl.ds(off, _C)]], rb, sem)

    def wait_tile(rb, sem):
        pltpu.make_async_copy(w_pos.at[pl.ds(0, _C)], rb, sem).wait()

    pltpu.sync_copy(ids_hbm.at[:, pl.ds(pos0, _POS_PER_W)], idx_v)

    def load_pos(g, pb, sem):
        pltpu.async_copy(w_pos.at[pl.ds(pos0 + g * _C, _C)], pb, sem)

    pltpu.async_copy(w_pos.at[pl.ds(pos0, _C)], rbuf1, sem_s1)
    load_pos(0, posbuf0, sem_p0)
    gather(0, 0, rbuf0, sem_g0)

    def do_chunk(g, pb):
        for b in range(_BATCH):
            cur, nxt = b % 2, 1 - b % 2

            wait_tile(rbufs[nxt], ssems[nxt])

            if b < _BATCH - 1:
                gather(b + 1, g * _C, rbufs[nxt], gsems[nxt])
            else:
                gather(0, lax.rem((g + 1) * _C, _POS_PER_W),
                       rbufs[nxt], gsems[nxt])

            wait_tile(rbufs[cur], gsems[cur])

            def add_row(r, carry):
                for j in range(_NVREG):
                    s = pl.ds(j * 16, 16)
                    rbufs[cur][r, s] = rbufs[cur][r, s] + pb[r, s]
                return carry

            lax.fori_loop(0, _C, add_row, 0, unroll=False)

            pltpu.async_copy(
                rbufs[cur],
                out_hbm.at[pl.ds(b * _SEQ + pos0 + g * _C, _C)],
                ssems[cur])

    def pair_body(k, _):
        g0 = 2 * k
        load_pos(g0 + 1, posbuf1, sem_p1)
        wait_tile(posbuf0, sem_p0)
        do_chunk(g0, posbuf0)
        load_pos(lax.rem(g0 + 2, _NCHUNK), posbuf0, sem_p0)
        wait_tile(posbuf1, sem_p1)
        do_chunk(g0 + 1, posbuf1)
        return _

    lax.fori_loop(0, _NCHUNK // 2, pair_body, 0, unroll=False)

    wait_tile(rbuf0, sem_g0)
    wait_tile(posbuf0, sem_p0)
    wait_tile(rbuf1, sem_s1)


@jax.jit
def _embed(input_ids, w_word, w_pos):
    mesh = plsc.VectorSubcoreMesh(core_axis_name="c", subcore_axis_name="s")
    k = pl.kernel(
        _body,
        out_type=jax.ShapeDtypeStruct((_BATCH * _SEQ, _EMBED), jnp.float32),
        mesh=mesh,
        scratch_types=[
            pltpu.VMEM((_BATCH, _POS_PER_W), jnp.int32),
            pltpu.VMEM((_C, _EMBED), jnp.float32),
            pltpu.VMEM((_C, _EMBED), jnp.float32),
            pltpu.VMEM((_C, _EMBED), jnp.float32),
            pltpu.VMEM((_C, _EMBED), jnp.float32),
            pltpu.SemaphoreType.DMA,
            pltpu.SemaphoreType.DMA,
            pltpu.SemaphoreType.DMA,
            pltpu.SemaphoreType.DMA,
            pltpu.SemaphoreType.DMA,
            pltpu.SemaphoreType.DMA,
        ],
    )
    return k(input_ids, w_word, w_pos)


def kernel(input_ids, W_word, W_pos):
    ids = input_ids.astype(jnp.int32)
    out = _embed(ids, W_word, W_pos)
    return out.reshape(_BATCH, _SEQ, _EMBED)


# speedup vs baseline: 2.4313x; 1.0111x over previous
import jax
import jax.numpy as jnp
from jax import lax
from jax.experimental import pallas as pl
from jax.experimental.pallas import tpu as pltpu
from jax.experimental.pallas import tpu_sc as plsc

_VOCAB = 50304
_MAX_POS = 8192
_EMBED = 768
_BATCH = 4
_SEQ = 8192

_NC = 2
_NS = 16
_NW = _NC * _NS
_POS_PER_W = _SEQ // _NW
_C = 32
_NCHUNK = _POS_PER_W // _C
_NVREG = _EMBED // 16


def _body(ids_hbm, w_word, w_pos, out_hbm,
          idx_v, rbuf0, rbuf1, rbuf2, rbuf3, posbuf,
          sem_g0, sem_g1, sem_g2, sem_g3,
          sem_s0, sem_s1, sem_s2, sem_s3, sem_p):
    cid = lax.axis_index("c")
    sid = lax.axis_index("s")
    wid = sid * _NC + cid
    pos0 = wid * _POS_PER_W

    rbufs = (rbuf0, rbuf1, rbuf2, rbuf3)
    gsems = (sem_g0, sem_g1, sem_g2, sem_g3)
    ssems = (sem_s0, sem_s1, sem_s2, sem_s3)

    def gather(b_idx, off, rb, sem):
        pltpu.async_copy(w_word.at[idx_v.at[b_idx, pl.ds(off, _C)]], rb, sem)

    def wait_tile(rb, sem):
        pltpu.make_async_copy(w_pos.at[pl.ds(0, _C)], rb, sem).wait()

    def load_pos(g, sem):
        pltpu.async_copy(w_pos.at[pl.ds(pos0 + g * _C, _C)], posbuf, sem)

    pltpu.sync_copy(ids_hbm.at[:, pl.ds(pos0, _POS_PER_W)], idx_v)

    pltpu.async_copy(w_pos.at[pl.ds(pos0, _C)], rbuf2, sem_s2)
    pltpu.async_copy(w_pos.at[pl.ds(pos0, _C)], rbuf3, sem_s3)
    load_pos(0, sem_p)
    gather(0, 0, rbuf0, sem_g0)
    gather(1, 0, rbuf1, sem_g1)

    def chunk_body(g, _):
        gnxt = lax.rem(g + 1, _NCHUNK)
        for b in range(_BATCH):
            n2 = (b + 2) % _BATCH

            wait_tile(rbufs[n2], ssems[n2])

            if b < 2:
                gather(b + 2, g * _C, rbufs[n2], gsems[n2])
            else:
                gather(b - 2, gnxt * _C, rbufs[n2], gsems[n2])

            if b == 0:
                wait_tile(posbuf, sem_p)

            wait_tile(rbufs[b], gsems[b])

            def add_row(r, carry):
                for j in range(_NVREG):
                    s = pl.ds(j * 16, 16)
                    rbufs[b][r, s] = rbufs[b][r, s] + posbuf[r, s]
                return carry

            lax.fori_loop(0, _C, add_row, 0, unroll=False)

            if b == _BATCH - 1:
                load_pos(gnxt, sem_p)

            pltpu.async_copy(
                rbufs[b],
                out_hbm.at[pl.ds(b * _SEQ + pos0 + g * _C, _C)],
                ssems[b])
        return _

    lax.fori_loop(0, _NCHUNK, chunk_body, 0, unroll=False)

    wait_tile(rbuf0, sem_g0)
    wait_tile(rbuf1, sem_g1)
    wait_tile(posbuf, sem_p)
    wait_tile(rbuf2, sem_s2)
    wait_tile(rbuf3, sem_s3)


@jax.jit
def _embed(input_ids, w_word, w_pos):
    mesh = plsc.VectorSubcoreMesh(core_axis_name="c", subcore_axis_name="s")
    k = pl.kernel(
        _body,
        out_type=jax.ShapeDtypeStruct((_BATCH * _SEQ, _EMBED), jnp.float32),
        mesh=mesh,
        scratch_types=[
            pltpu.VMEM((_BATCH, _POS_PER_W), jnp.int32),
            pltpu.VMEM((_C, _EMBED), jnp.float32),
            pltpu.VMEM((_C, _EMBED), jnp.float32),
            pltpu.VMEM((_C, _EMBED), jnp.float32),
            pltpu.VMEM((_C, _EMBED), jnp.float32),
            pltpu.VMEM((_C, _EMBED), jnp.float32),
            pltpu.SemaphoreType.DMA,
            pltpu.SemaphoreType.DMA,
            pltpu.SemaphoreType.DMA,
            pltpu.SemaphoreType.DMA,
            pltpu.SemaphoreType.DMA,
            pltpu.SemaphoreType.DMA,
            pltpu.SemaphoreType.DMA,
            pltpu.SemaphoreType.DMA,
            pltpu.SemaphoreType.DMA,
        ],
    )
    return k(input_ids, w_word, w_pos)


def kernel(input_ids, W_word, W_pos):
    ids = input_ids.astype(jnp.int32)
    out = _embed(ids, W_word, W_pos)
    return out.reshape(_BATCH, _SEQ, _EMBED)
